# baseline (speedup 1.0000x reference)
import jax
import jax.numpy as jnp
from jax.experimental import pallas as pl

N = 512
B = 64
L = 12
U = 64
K = 2
NUM_MAT = K + 1
BC = 16
SUB = 128


def _dcgru_chunk(xs, hs, s, wg, bg, wc, bc, pad):
    zpad = jnp.zeros((N, pad), jnp.bfloat16) if pad else None

    def panel(states):
        parts = []
        for i in range(BC):
            parts.append(states[i])
            parts.append(xs[i])
            if pad:
                parts.append(zpad)
        return jnp.concatenate(parts, axis=1)

    def diffuse(p0):
        p1 = jnp.dot(s, p0, preferred_element_type=jnp.float32).astype(jnp.bfloat16)
        p2 = jnp.dot(s, p1, preferred_element_type=jnp.float32).astype(jnp.bfloat16)
        return p1, p2

    def proj(p0, p1, p2, w, bias, i):
        sl = slice(i * SUB, (i + 1) * SUB)
        pk = jnp.concatenate([p0[:, sl], p1[:, sl], p2[:, sl]], axis=1)
        return jnp.dot(pk, w, preferred_element_type=jnp.float32) + bias

    g0 = panel([h.astype(jnp.bfloat16) for h in hs])
    g1, g2 = diffuse(g0)
    rs, us = [], []
    for i in range(BC):
        val = jax.nn.sigmoid(proj(g0, g1, g2, wg, bg, i))
        rs.append(val[:, :U])
        us.append(val[:, U:])

    c0 = panel([(rs[i] * hs[i]).astype(jnp.bfloat16) for i in range(BC)])
    c1, c2 = diffuse(c0)
    outs = []
    for i in range(BC):
        c = jnp.tanh(proj(c0, c1, c2, wc, bc, i))
        outs.append(us[i] * hs[i] + (1.0 - us[i]) * c)
    return outs


def _body(x_ref, h0_ref, h1_ref, s_ref,
          wg0_ref, bg0_ref, wc0_ref, bc0_ref,
          wg1_ref, bg1_ref, wc1_ref, bc1_ref,
          hid_ref):
    s = s_ref[...]

    xs0 = [x_ref[i].astype(jnp.bfloat16) for i in range(BC)]
    hs0 = [h0_ref[i] for i in range(BC)]
    h0n = _dcgru_chunk(xs0, hs0, s, wg0_ref[...], bg0_ref[...],
                       wc0_ref[...], bc0_ref[...], SUB - (L + U))

    xs1 = [h.astype(jnp.bfloat16) for h in h0n]
    hs1 = [h1_ref[i] for i in range(BC)]
    h1n = _dcgru_chunk(xs1, hs1, s, wg1_ref[...], bg1_ref[...],
                       wc1_ref[...], bc1_ref[...], SUB - (U + U))

    for i in range(BC):
        hid_ref[0, i] = h0n[i]
        hid_ref[1, i] = h1n[i]


def _fold_weights(W, F, out):
    in_sz = F + U
    W3 = W.reshape(in_sz, NUM_MAT, out).transpose(1, 0, 2)
    W3 = jnp.stack([W3[0] - W3[2], W3[1], 2.0 * W3[2]], axis=0)
    W3 = jnp.concatenate([W3[:, F:, :], W3[:, :F, :],
                          jnp.zeros((NUM_MAT, SUB - in_sz, out), W3.dtype)],
                         axis=1)
    return W3.reshape(NUM_MAT * SUB, out).astype(jnp.bfloat16)


@jax.jit
def kernel(inputs, hidden_state, support, Wg0, bg0, Wc0, bc0, Wg1, bg1, Wc1, bc1):
    x = inputs.reshape(B, N, L)
    h0_in = hidden_state[0].reshape(B, N, U)
    h1_in = hidden_state[1].reshape(B, N, U)
    s16 = support.astype(jnp.bfloat16)
    args = (x, h0_in, h1_in, s16,
            _fold_weights(Wg0, L, 2 * U), bg0.reshape(1, 2 * U),
            _fold_weights(Wc0, L, U), bc0.reshape(1, U),
            _fold_weights(Wg1, U, 2 * U), bg1.reshape(1, 2 * U),
            _fold_weights(Wc1, U, U), bc1.reshape(1, U))

    const = lambda b: (0, 0)
    const3 = lambda b: (0, 0, 0)
    hid = pl.pallas_call(
        _body,
        grid=(B // BC,),
        in_specs=[
            pl.BlockSpec((BC, N, L), lambda b: (b, 0, 0)),
            pl.BlockSpec((BC, N, U), lambda b: (b, 0, 0)),
            pl.BlockSpec((BC, N, U), lambda b: (b, 0, 0)),
            pl.BlockSpec((N, N), const),
            pl.BlockSpec((NUM_MAT * SUB, 2 * U), const),
            pl.BlockSpec((1, 2 * U), const),
            pl.BlockSpec((NUM_MAT * SUB, U), const),
            pl.BlockSpec((1, U), const),
            pl.BlockSpec((NUM_MAT * SUB, 2 * U), const),
            pl.BlockSpec((1, 2 * U), const),
            pl.BlockSpec((NUM_MAT * SUB, U), const),
            pl.BlockSpec((1, U), const),
        ],
        out_specs=pl.BlockSpec((2, BC, N, U), lambda b: (0, b, 0, 0)),
        out_shape=jax.ShapeDtypeStruct((2, B, N, U), jnp.float32),
    )(*args)
    hid = hid.reshape(2, B, N * U)
    return hid[1], hid
